# baseline (speedup 1.0000x reference)
import math

import jax
import jax.numpy as jnp
from jax.experimental import pallas as pl
from jax.experimental.pallas import tpu as pltpu

DIM = 256
DEPTH = 6
HEADS = 8
HD = DIM // HEADS
MLP = 1024
E = 8
TOPK = 2
VOCAB = 8000
B = 8
T = 128
HP = 4
BT = B * T
VPAD = 8192
BF = jnp.bfloat16


def _erf(x):
    a1, a2, a3, a4, a5 = (0.254829592, -0.284496736, 1.421413741,
                          -1.453152027, 1.061405429)
    p = 0.3275911
    s = jnp.sign(x)
    ax = jnp.abs(x)
    t = 1.0 / (1.0 + p * ax)
    y = 1.0 - ((((a5 * t + a4) * t + a3) * t + a2) * t + a1) * t * jnp.exp(-ax * ax)
    return s * y


def _gelu(x):
    return 0.5 * x * (1.0 + _erf(x * (1.0 / math.sqrt(2.0))))


def _ln(x, g, b, eps=1e-5):
    m = jnp.mean(x, axis=-1, keepdims=True)
    v = jnp.mean((x - m) * (x - m), axis=-1, keepdims=True)
    return (x - m) * jax.lax.rsqrt(v + eps) * g + b


def _dot_bf(a, wb):
    return jnp.dot(a, wb, preferred_element_type=jnp.float32)


def _mm_gelu_body(p_ref, w_ref, b_ref, o_ref, *, c, exact):
    if exact:
        acc = jnp.dot(p_ref[...], w_ref[...],
                      preferred_element_type=jnp.float32,
                      precision=jax.lax.Precision.HIGHEST)
    else:
        acc = jnp.dot(p_ref[:, 0:c], w_ref[0:c, :],
                      preferred_element_type=jnp.float32)
        for j in range(1, 9):
            acc += jnp.dot(p_ref[:, j * c:(j + 1) * c],
                           w_ref[j * c:(j + 1) * c, :],
                           preferred_element_type=jnp.float32)
    o_ref[...] = _gelu(acc + b_ref[...])


def _mm_gelu(patches, wb, bias, c, bm=2048):
    import functools
    m, k = patches.shape
    n = wb.shape[1]
    bm = min(bm, m)
    grid = (pl.cdiv(m, bm),)
    return pl.pallas_call(
        functools.partial(_mm_gelu_body, c=c, exact=(c == 1)),
        grid=grid,
        in_specs=[
            pl.BlockSpec((bm, k), lambda i: (i, 0)),
            pl.BlockSpec((k, n), lambda i: (0, 0)),
            pl.BlockSpec((1, n), lambda i: (0, 0)),
        ],
        out_specs=pl.BlockSpec((bm, n), lambda i: (i, 0)),
        out_shape=jax.ShapeDtypeStruct((m, n), jnp.float32),
    )(patches, wb, bias)


def _patches(x, s):
    x = jnp.pad(x, ((0, 0), (1, 1), (1, 1), (0, 0)))
    n, h, w, c = x.shape
    oh = (h - 3) // s + 1
    ow = (w - 3) // s + 1
    cols = []
    for di in range(3):
        for dj in range(3):
            cols.append(jax.lax.slice(
                x, (0, di, dj, 0),
                (n, di + s * (oh - 1) + 1, dj + s * (ow - 1) + 1, c),
                (1, s, s, 1)))
    p = jnp.stack(cols, axis=3)
    return p.reshape(n * oh * ow, 9 * c), (n, oh, ow)


def _stem(x, sp):
    y = x.transpose(0, 2, 3, 1)
    for name, stride in (("0", 2), ("1", 2), ("2", 2), ("3", 1)):
        w = sp["w" + name]
        o = w.shape[0]
        c = w.shape[1]
        p, (n, oh, ow) = _patches(y, stride)
        wb = w.transpose(2, 3, 1, 0).reshape(9 * c, o)
        bias = sp["b" + name].reshape(1, o)
        y = _mm_gelu(p, wb, bias, c).reshape(n, oh, ow, o)
    return y


def _pos_emb():
    pos = jnp.arange(T, dtype=jnp.float32)[:, None]
    div = jnp.exp(jnp.arange(0, DIM, 2, dtype=jnp.float32)
                  * (-math.log(10000.0) / DIM))
    pe = jnp.zeros((T, DIM), jnp.float32)
    pe = pe.at[:, 0::2].set(jnp.sin(pos * div))
    pe = pe.at[:, 1::2].set(jnp.cos(pos * div))
    return pe[None]


def _blocks_body(tok_ref, ln1g_ref, ln1b_ref, inw_ref, inb_ref,
                 outw_ref, outb_ref, ln2g_ref, ln2b_ref, rw_ref,
                 rb_ref, w1_ref, b1_ref, w2_ref, b2_ref,
                 xout_ref, imp_ref, zs, macc, cs):
    l = pl.program_id(0)
    e = pl.program_id(1)

    @pl.when(jnp.logical_and(l == 0, e == 0))
    def _():
        for b in range(B):
            xout_ref[pl.ds(b * T, T)] = jnp.mean(tok_ref[b], axis=0)

    @pl.when(e == 0)
    def _():
        scale = 1.0 / math.sqrt(HD)
        xall = xout_ref[...]
        yall = _ln(xall, ln1g_ref[0], ln1b_ref[0])
        qkv = _dot_bf(yall, inw_ref[0]) + inb_ref[0]
        rowh = jax.lax.broadcasted_iota(jnp.int32, (HEADS * T, DIM), 0) // T
        laneh = jax.lax.broadcasted_iota(jnp.int32, (HEADS * T, DIM), 1) // HD
        hmask = (rowh == laneh).astype(jnp.float32)
        oall = []
        for b in range(B):
            q = qkv[b * T:(b + 1) * T, 0:DIM]
            k = qkv[b * T:(b + 1) * T, DIM:2 * DIM]
            v = qkv[b * T:(b + 1) * T, 2 * DIM:3 * DIM]
            kb = jnp.concatenate([k] * HEADS, axis=0) * hmask
            vb = jnp.concatenate([v] * HEADS, axis=0) * hmask
            s_all = jax.lax.dot_general(
                q, kb, (((1,), (1,)), ((), ())),
                preferred_element_type=jnp.float32) * scale
            ps = []
            for h in range(HEADS):
                s = s_all[:, T * h:T * (h + 1)]
                s = s - jnp.max(s, axis=-1, keepdims=True)
                p = jnp.exp(s)
                ps.append(p / jnp.sum(p, axis=-1, keepdims=True))
            p_all = jnp.concatenate(ps, axis=-1)
            oall.append(jnp.dot(p_all, vb,
                                preferred_element_type=jnp.float32))
        o = jnp.concatenate(oall, axis=0)
        attn = _dot_bf(o, outw_ref[0]) + outb_ref[0]
        xall = xall + attn
        xout_ref[...] = xall
        z = _ln(xall, ln2g_ref[0], ln2b_ref[0])
        zs[...] = z

        rl = _dot_bf(z, rw_ref[0]) + rb_ref[0]
        pm = jnp.max(rl, axis=-1, keepdims=True)
        pe_ = jnp.exp(rl - pm)
        probs = pe_ / jnp.sum(pe_, axis=-1, keepdims=True)
        imp_ref[0] = jnp.sum(probs, axis=0, keepdims=True)

        lane = jax.lax.broadcasted_iota(jnp.int32, (BT, E), 1)
        m1 = jnp.max(rl, axis=-1, keepdims=True)
        i1 = jnp.min(jnp.where(rl == m1, lane, E), axis=-1, keepdims=True)
        oh1 = lane == i1
        rl2 = jnp.where(oh1, -jnp.inf, rl)
        m2 = jnp.max(rl2, axis=-1, keepdims=True)
        i2 = jnp.min(jnp.where(rl2 == m2, lane, E), axis=-1, keepdims=True)
        oh2 = lane == i2
        d = jnp.exp(m2 - m1)
        g1 = 1.0 / (1.0 + d)
        g2 = d / (1.0 + d)
        cs[...] = jnp.where(oh1, g1, 0.0) + jnp.where(oh2, g2, 0.0)
        macc[...] = xout_ref[...]

    z = zs[...]
    h = _gelu(_dot_bf(z, w1_ref[0, 0]) + b1_ref[0, 0])
    eo = _dot_bf(h, w2_ref[0, 0]) + b2_ref[0, 0]
    sel = jax.lax.broadcasted_iota(jnp.int32, (BT, E), 1) == e
    col = jnp.sum(jnp.where(sel, cs[...], 0.0), axis=-1, keepdims=True)
    macc[...] += col * eo

    @pl.when(e == E - 1)
    def _():
        xout_ref[...] = macc[...]


def _blocks(tokens4, stk):
    grid = (DEPTH, E)
    l_only = lambda l, e: (l, 0, 0)
    le = lambda l, e: (l, e, 0, 0)
    return pl.pallas_call(
        _blocks_body,
        grid=grid,
        in_specs=[
            pl.BlockSpec((B, HP, T, DIM), lambda l, e: (0, 0, 0, 0)),
            pl.BlockSpec((1, 1, DIM), l_only),
            pl.BlockSpec((1, 1, DIM), l_only),
            pl.BlockSpec((1, DIM, 3 * DIM), l_only),
            pl.BlockSpec((1, 1, 3 * DIM), l_only),
            pl.BlockSpec((1, DIM, DIM), l_only),
            pl.BlockSpec((1, 1, DIM), l_only),
            pl.BlockSpec((1, 1, DIM), l_only),
            pl.BlockSpec((1, 1, DIM), l_only),
            pl.BlockSpec((1, DIM, E), l_only),
            pl.BlockSpec((1, 1, E), l_only),
            pl.BlockSpec((1, 1, DIM, MLP), le),
            pl.BlockSpec((1, 1, 1, MLP), le),
            pl.BlockSpec((1, 1, MLP, DIM), le),
            pl.BlockSpec((1, 1, 1, DIM), le),
        ],
        out_specs=[
            pl.BlockSpec((BT, DIM), lambda l, e: (0, 0)),
            pl.BlockSpec((1, 1, E), l_only),
        ],
        out_shape=[
            jax.ShapeDtypeStruct((BT, DIM), jnp.float32),
            jax.ShapeDtypeStruct((DEPTH, 1, E), jnp.float32),
        ],
        scratch_shapes=[
            pltpu.VMEM((BT, DIM), jnp.float32),
            pltpu.VMEM((BT, DIM), jnp.float32),
            pltpu.VMEM((BT, E), jnp.float32),
        ],
    )(tokens4, *stk)


def _head_body(x_ref, g_ref, b_ref, w_ref, hb_ref, o_ref):
    z = _ln(x_ref[...], g_ref[...], b_ref[...])
    o_ref[...] = _dot_bf(z, w_ref[...]) + hb_ref[...]


def _head(xf, g, b, wb, hb, bn=1024):
    m = xf.shape[0]
    grid = (VPAD // bn,)
    return pl.pallas_call(
        _head_body,
        grid=grid,
        in_specs=[
            pl.BlockSpec((m, DIM), lambda i: (0, 0)),
            pl.BlockSpec((1, DIM), lambda i: (0, 0)),
            pl.BlockSpec((1, DIM), lambda i: (0, 0)),
            pl.BlockSpec((DIM, bn), lambda i: (0, i)),
            pl.BlockSpec((1, bn), lambda i: (0, i)),
        ],
        out_specs=pl.BlockSpec((m, bn), lambda i: (0, i)),
        out_shape=jax.ShapeDtypeStruct((m, VPAD), jnp.float32),
    )(xf, g, b, wb, hb)


def kernel(x, params):
    blocks = params["blocks"]

    tokens4 = _stem(x, params["stem"]) + _pos_emb()[None]

    def stack(f):
        return jnp.stack([f(bp) for bp in blocks])

    def stack_bf(f):
        return jnp.stack([f(bp) for bp in blocks])

    stk = [
        stack(lambda p: p["ln1_g"].reshape(1, DIM)),
        stack(lambda p: p["ln1_b"].reshape(1, DIM)),
        stack_bf(lambda p: p["in_w"].T),
        stack(lambda p: p["in_b"].reshape(1, 3 * DIM)),
        stack_bf(lambda p: p["out_w"].T),
        stack(lambda p: p["out_b"].reshape(1, DIM)),
        stack(lambda p: p["ln2_g"].reshape(1, DIM)),
        stack(lambda p: p["ln2_b"].reshape(1, DIM)),
        stack_bf(lambda p: p["router_w"].T),
        stack(lambda p: p["router_b"].reshape(1, E)),
        stack_bf(lambda p: p["w1"].transpose(0, 2, 1)),
        stack(lambda p: p["b1"].reshape(E, 1, MLP)),
        stack_bf(lambda p: p["w2"].transpose(0, 2, 1)),
        stack(lambda p: p["b2"].reshape(E, 1, DIM)),
    ]

    xf, imp = _blocks(tokens4, stk)

    head_wt = jnp.zeros((DIM, VPAD), jnp.float32).at[:, :VOCAB].set(
        params["head_w"].T)
    head_b = jnp.zeros((1, VPAD), jnp.float32).at[:, :VOCAB].set(
        params["head_b"][None])
    logits = _head(xf, params["ln_g"].reshape(1, DIM),
                   params["ln_b"].reshape(1, DIM), head_wt, head_b)
    logits = logits[:, :VOCAB].reshape(B, T, VOCAB)

    imp_m = imp[:, 0, :] / BT
    aux = jnp.mean((imp_m - 1.0 / E) ** 2, axis=-1)
    aux_total = jnp.sum(aux) / DEPTH
    return logits, aux_total.astype(jnp.float32)
